# baseline (speedup 1.0000x reference)
import jax
import jax.numpy as jnp
from jax.experimental import pallas as pl
from jax.experimental.pallas import tpu as pltpu

_DBLK = 32


def _body(t_ref, o_ref, s0, s1, sems):
    nblk = t_ref.shape[0] // _DBLK
    half = nblk // 2
    b = o_ref.shape[0]
    h, w = o_ref.shape[2], o_ref.shape[3]
    bufs = (s0, s1)
    pid = pl.program_id(0)

    def copies(blk):
        s = bufs[blk % 2]
        return [
            pltpu.make_async_copy(
                s, o_ref.at[bb, pl.ds(blk * _DBLK, _DBLK)], sems.at[blk % 2, bb]
            )
            for bb in range(b)
        ]

    def run_half(is_row):
        blk0 = half * is_row
        for k in range(half):
            blk = blk0 + k
            s = bufs[blk % 2]
            if k >= 2:
                for cp in copies(blk - 2):
                    cp.wait()
            t = t_ref[pl.ds(blk * _DBLK, _DBLK), :]
            if is_row:
                s[...] = jnp.broadcast_to(t[:, :, None], (_DBLK, h, w))
            else:
                s[...] = jnp.broadcast_to(t[:, None, :], (_DBLK, h, w))
            for cp in copies(blk):
                cp.start()
        for blk in (blk0 + half - 2, blk0 + half - 1):
            for cp in copies(blk):
                cp.wait()

    @pl.when(pid == 0)
    def _col():
        run_half(0)

    @pl.when(pid == 1)
    def _row():
        run_half(1)


def kernel(x, row_embed, col_embed):
    b = x.shape[0]
    h, w = x.shape[2], x.shape[3]
    d_half = row_embed.shape[1]
    d = 2 * d_half
    t = jnp.concatenate([col_embed[:w].T, row_embed[:h].T], axis=0)

    return pl.pallas_call(
        _body,
        grid=(2,),
        in_specs=[pl.BlockSpec(memory_space=pltpu.VMEM)],
        out_specs=pl.BlockSpec(memory_space=pl.ANY),
        out_shape=jax.ShapeDtypeStruct((b, d, h, w), x.dtype),
        scratch_shapes=[
            pltpu.VMEM((_DBLK, h, w), jnp.float32),
            pltpu.VMEM((_DBLK, h, w), jnp.float32),
            pltpu.SemaphoreType.DMA((2, b)),
        ],
        compiler_params=pltpu.CompilerParams(
            dimension_semantics=("parallel",),
        ),
    )(t)
